# baseline (speedup 1.0000x reference)
import functools

import jax
import jax.numpy as jnp
from jax import lax
from jax.experimental import pallas as pl
from jax.experimental.pallas import tpu as pltpu
from jax.experimental.pallas import tpu_sc as plsc

B = 16384
D = 32
V = 100001
LBLK = 1024
NQB = 26
Q = NQB * LBLK
NVBLK = -(-V // LBLK)

_info = plsc.get_sparse_core_info()
_NC = _info.num_cores
_NS = _info.num_subcores
_NW = _NC * _NS
_BPW = B // _NW

_mesh = plsc.VectorSubcoreMesh(core_axis_name="c", subcore_axis_name="s")

_DN_T = (((0,), (0,)), ((), ()))
_DN = (((1,), (0,)), ((), ()))


def _t1_body(x0, x1, x2, x3, w1_ref, b1_ref, w2_ref, b2_ref, o_ref):
    w1 = w1_ref[...]
    b1 = b1_ref[...]
    w2 = w2_ref[...]
    b2 = b2_ref[...]
    x = jnp.concatenate(
        [x0[...], x1[...], x2[...], x3[...]], axis=1
    )
    h = jnp.maximum(
        lax.dot_general(x, w1, _DN_T, preferred_element_type=jnp.float32)
        + b1,
        0.0,
    )
    ot = (
        lax.dot_general(h, w2, _DN, preferred_element_type=jnp.float32)
        + b2
    )
    o_ref[...] = jnp.concatenate(
        [ot[c * LBLK:(c + 1) * LBLK] for c in range(4)], axis=1
    )


def _t1(tableT, W1, b1, W2, b2):
    def tmap(c):
        return lambda i: (0, jnp.minimum(NQB * c + i, NVBLK - 1))

    return pl.pallas_call(
        _t1_body,
        grid=(NQB,),
        in_specs=[
            pl.BlockSpec((D, LBLK), tmap(0)),
            pl.BlockSpec((D, LBLK), tmap(1)),
            pl.BlockSpec((D, LBLK), tmap(2)),
            pl.BlockSpec((D, LBLK), tmap(3)),
            pl.BlockSpec(W1.shape, lambda i: (0, 0)),
            pl.BlockSpec((1, W1.shape[1]), lambda i: (0, 0)),
            pl.BlockSpec(W2.shape, lambda i: (0, 0)),
            pl.BlockSpec((1, W2.shape[1]), lambda i: (0, 0)),
        ],
        out_specs=pl.BlockSpec((LBLK, 4 * D), lambda i: (i, 0)),
        out_shape=jax.ShapeDtypeStruct((Q, 4 * D), jnp.float32),
    )(
        tableT, tableT, tableT, tableT,
        W1, b1.reshape(1, -1), W2, b2.reshape(1, -1),
    )


@functools.partial(
    pl.kernel,
    mesh=_mesh,
    out_type=jax.ShapeDtypeStruct((D, B), jnp.float32),
    scratch_types=[
        pltpu.VMEM((_BPW,), jnp.int32),
        pltpu.VMEM((_BPW,), jnp.int32),
        pltpu.VMEM((_BPW,), jnp.int32),
        pltpu.VMEM((_BPW, 4 * D), jnp.float32),
        pltpu.VMEM((D, _BPW), jnp.float32),
        pltpu.SemaphoreType.DMA,
    ],
    compiler_params=pltpu.CompilerParams(needs_layout_passes=False),
)
def _sc_gather(lines_hbm, idx_hbm, out_hbm, idx_v, j_v, col_v, rows_v,
               outT_v, sem):
    wid = lax.axis_index("s") * _NC + lax.axis_index("c")
    base = wid * _BPW
    pltpu.sync_copy(idx_hbm.at[pl.ds(base, _BPW)], idx_v)
    for k in range(_BPW // 16):
        sl = pl.ds(k * 16, 16)
        v = idx_v[sl]
        slot = lax.div(v, Q)
        j_v[sl] = v - slot * Q
        col_v[sl] = slot * D
    pltpu.async_copy(lines_hbm.at[j_v], rows_v, sem).wait()

    row_iota = lax.iota(jnp.int32, 16)
    for k in range(_BPW // 16):
        rsl = pl.ds(k * 16, 16)
        rows16 = row_iota + k * 16
        col0 = col_v[rsl]
        for d in range(D):
            vals = plsc.load_gather(rows_v, [rows16, col0 + d])
            outT_v[d, rsl] = vals
    pltpu.sync_copy(outT_v, out_hbm.at[:, pl.ds(base, _BPW)])


def kernel(user_id, table, W1, b1, W2, b2):
    uid = user_id.astype(jnp.int32)
    lines = _t1(table.T, W1, b1, W2, b2)
    return _sc_gather(lines, uid).T


# speedup vs baseline: 1.0388x; 1.0388x over previous
import functools

import jax
import jax.numpy as jnp
from jax import lax
from jax.experimental import pallas as pl
from jax.experimental.pallas import tpu as pltpu
from jax.experimental.pallas import tpu_sc as plsc

B = 16384
D = 32
V = 100001
LBLK = 4096
NQB = 7
Q = NQB * LBLK
NVBLK = -(-V // LBLK)

_info = plsc.get_sparse_core_info()
_NC = _info.num_cores
_NS = _info.num_subcores
_NW = _NC * _NS
_BPW = B // _NW

_mesh = plsc.VectorSubcoreMesh(core_axis_name="c", subcore_axis_name="s")

_DN_T = (((0,), (0,)), ((), ()))
_DN = (((1,), (0,)), ((), ()))


def _t1_body(x0, x1, x2, x3, w1_ref, b1_ref, w2_ref, b2_ref, o_ref):
    w1 = w1_ref[...]
    b1 = b1_ref[...]
    w2 = w2_ref[...]
    b2 = b2_ref[...]
    x = jnp.concatenate(
        [x0[...], x1[...], x2[...], x3[...]], axis=1
    )
    h = jnp.maximum(
        lax.dot_general(x, w1, _DN_T, preferred_element_type=jnp.float32)
        + b1,
        0.0,
    )
    ot = (
        lax.dot_general(h, w2, _DN, preferred_element_type=jnp.float32)
        + b2
    )
    o_ref[...] = jnp.concatenate(
        [ot[c * LBLK:(c + 1) * LBLK] for c in range(4)], axis=1
    )


def _t1(tableT, W1, b1, W2, b2):
    def tmap(c):
        return lambda i: (0, jnp.minimum(NQB * c + i, NVBLK - 1))

    return pl.pallas_call(
        _t1_body,
        grid=(NQB,),
        in_specs=[
            pl.BlockSpec((D, LBLK), tmap(0)),
            pl.BlockSpec((D, LBLK), tmap(1)),
            pl.BlockSpec((D, LBLK), tmap(2)),
            pl.BlockSpec((D, LBLK), tmap(3)),
            pl.BlockSpec(W1.shape, lambda i: (0, 0)),
            pl.BlockSpec((1, W1.shape[1]), lambda i: (0, 0)),
            pl.BlockSpec(W2.shape, lambda i: (0, 0)),
            pl.BlockSpec((1, W2.shape[1]), lambda i: (0, 0)),
        ],
        out_specs=pl.BlockSpec((LBLK, 4 * D), lambda i: (i, 0)),
        out_shape=jax.ShapeDtypeStruct((Q, 4 * D), jnp.float32),
    )(
        tableT, tableT, tableT, tableT,
        W1, b1.reshape(1, -1), W2, b2.reshape(1, -1),
    )


@functools.partial(
    pl.kernel,
    mesh=_mesh,
    out_type=jax.ShapeDtypeStruct((D, B), jnp.float32),
    scratch_types=[
        pltpu.VMEM((_BPW,), jnp.int32),
        pltpu.VMEM((_BPW,), jnp.int32),
        pltpu.VMEM((_BPW,), jnp.int32),
        pltpu.VMEM((_BPW, 4 * D), jnp.float32),
        pltpu.VMEM((D, _BPW), jnp.float32),
        pltpu.SemaphoreType.DMA,
    ],
    compiler_params=pltpu.CompilerParams(needs_layout_passes=False),
)
def _sc_gather(lines_hbm, idx_hbm, out_hbm, idx_v, j_v, col_v, rows_v,
               outT_v, sem):
    wid = lax.axis_index("s") * _NC + lax.axis_index("c")
    base = wid * _BPW
    pltpu.sync_copy(idx_hbm.at[pl.ds(base, _BPW)], idx_v)
    for k in range(_BPW // 16):
        sl = pl.ds(k * 16, 16)
        v = idx_v[sl]
        slot = lax.div(v, Q)
        j_v[sl] = v - slot * Q
        col_v[sl] = slot * D
    pltpu.async_copy(lines_hbm.at[j_v], rows_v, sem).wait()

    row_iota = lax.iota(jnp.int32, 16)
    for k in range(_BPW // 16):
        rsl = pl.ds(k * 16, 16)
        rows16 = row_iota + k * 16
        col0 = col_v[rsl]
        for d in range(D):
            vals = plsc.load_gather(rows_v, [rows16, col0 + d])
            outT_v[d, rsl] = vals
    pltpu.sync_copy(outT_v, out_hbm.at[:, pl.ds(base, _BPW)])


def kernel(user_id, table, W1, b1, W2, b2):
    uid = user_id.astype(jnp.int32)
    lines = _t1(table.T, W1, b1, W2, b2)
    return _sc_gather(lines, uid).T
